# baseline (speedup 1.0000x reference)
import jax
import jax.numpy as jnp
from jax.experimental import pallas as pl
from jax.experimental.pallas import tpu as pltpu
from jax.experimental.pallas import tpu_sc as plsc

T = 2048
C = 768
V = 50257
BV = 512
GATHER_WINDOW = 64


def _sc_gather(tok_embds, ids):
    mesh = plsc.VectorSubcoreMesh(core_axis_name="c", subcore_axis_name="s")
    nw = 32
    b_per_w = T // nw

    @pl.kernel(
        out_type=jax.ShapeDtypeStruct((T, C), tok_embds.dtype),
        mesh=mesh,
        scratch_types=[
            pltpu.VMEM((b_per_w,), jnp.int32),
            pltpu.VMEM((b_per_w, C), jnp.float32),
            pltpu.SemaphoreType.DMA,
        ],
    )
    def gather_kernel(table_hbm, idx_hbm, out_hbm, idx_v, rows_v, sem):
        wid = jax.lax.axis_index("s") * 2 + jax.lax.axis_index("c")
        base = wid * b_per_w
        pltpu.sync_copy(idx_hbm.at[pl.ds(base, b_per_w)], idx_v)
        pltpu.async_copy(table_hbm.at[idx_v], rows_v, sem).wait()
        pltpu.sync_copy(rows_v, out_hbm.at[pl.ds(base, b_per_w)])

    return gather_kernel(tok_embds, ids)


def _mm_body(x_ref, p_ref, w_ref, b_ref, o_ref, xb_ref):
    j = pl.program_id(0)

    @pl.when(j == 0)
    def _():
        xb_ref[...] = (x_ref[...] + p_ref[...]).astype(jnp.bfloat16)

    o_ref[...] = (
        jnp.dot(
            xb_ref[...],
            w_ref[...].astype(jnp.bfloat16),
            preferred_element_type=jnp.float32,
        )
        + b_ref[...]
    )


def _tc_matmul(x, pos, w, b2):
    nblk = pl.cdiv(V, BV)
    return pl.pallas_call(
        _mm_body,
        grid=(nblk,),
        in_specs=[
            pl.BlockSpec((T, C), lambda j: (0, 0)),
            pl.BlockSpec((T, C), lambda j: (0, 0)),
            pl.BlockSpec((C, BV), lambda j: (0, j)),
            pl.BlockSpec((1, BV), lambda j: (0, j)),
        ],
        out_specs=pl.BlockSpec((T, BV), lambda j: (0, j)),
        out_shape=jax.ShapeDtypeStruct((T, V), jnp.float32),
        scratch_shapes=[pltpu.VMEM((T, C), jnp.bfloat16)],
        compiler_params=pltpu.CompilerParams(dimension_semantics=("arbitrary",)),
    )(x, pos, w, b2)


def kernel(inputs, tok_embds, pos_embds, lm_head_w, lm_head_b):
    B, Tin = inputs.shape
    ids = inputs.reshape(B * Tin).astype(jnp.int32)
    x = _sc_gather(tok_embds, ids)
    logits = _tc_matmul(x, pos_embds, lm_head_w, lm_head_b.reshape(1, V))
    return logits.reshape(B, Tin, V)


# speedup vs baseline: 1.2014x; 1.2014x over previous
import jax
import jax.numpy as jnp
from jax.experimental import pallas as pl
from jax.experimental.pallas import tpu as pltpu
from jax.experimental.pallas import tpu_sc as plsc

T = 2048
C = 768
V = 50257
BV = 512


def _sc_gather(tok_embds, ids):
    mesh = plsc.VectorSubcoreMesh(core_axis_name="c", subcore_axis_name="s")
    nw = 32
    b_per_w = T // nw

    @pl.kernel(
        out_type=jax.ShapeDtypeStruct((T, C), tok_embds.dtype),
        mesh=mesh,
        scratch_types=[
            pltpu.VMEM((b_per_w,), jnp.int32),
            pltpu.VMEM((b_per_w, C), jnp.float32),
            pltpu.SemaphoreType.DMA,
        ],
    )
    def gather_kernel(table_hbm, idx_hbm, out_hbm, idx_v, rows_v, sem):
        wid = jax.lax.axis_index("s") * 2 + jax.lax.axis_index("c")
        base = wid * b_per_w
        pltpu.sync_copy(idx_hbm.at[pl.ds(base, b_per_w)], idx_v)
        pltpu.async_copy(table_hbm.at[idx_v], rows_v, sem).wait()
        pltpu.sync_copy(rows_v, out_hbm.at[pl.ds(base, b_per_w)])

    return gather_kernel(tok_embds, ids)


def _mm_body(x_ref, p_ref, wt_ref, b_ref, o_ref, xbt_ref):
    j = pl.program_id(0)

    @pl.when(j == 0)
    def _():
        xbt_ref[...] = jnp.transpose((x_ref[...] + p_ref[...]).astype(jnp.bfloat16))

    acc = jnp.dot(
        wt_ref[...].astype(jnp.bfloat16),
        xbt_ref[...],
        preferred_element_type=jnp.float32,
    )
    o_ref[...] = acc + jnp.transpose(b_ref[...])


def _tc_matmul_t(x, pos, wt, b2):
    nblk = pl.cdiv(V, BV)
    return pl.pallas_call(
        _mm_body,
        grid=(nblk,),
        in_specs=[
            pl.BlockSpec((T, C), lambda j: (0, 0)),
            pl.BlockSpec((T, C), lambda j: (0, 0)),
            pl.BlockSpec((BV, C), lambda j: (j, 0)),
            pl.BlockSpec((1, BV), lambda j: (0, j)),
        ],
        out_specs=pl.BlockSpec((BV, T), lambda j: (j, 0)),
        out_shape=jax.ShapeDtypeStruct((V, T), jnp.float32),
        scratch_shapes=[pltpu.VMEM((C, T), jnp.bfloat16)],
        compiler_params=pltpu.CompilerParams(dimension_semantics=("arbitrary",)),
    )(x, pos, wt, b2)


def kernel(inputs, tok_embds, pos_embds, lm_head_w, lm_head_b):
    B, Tin = inputs.shape
    ids = inputs.reshape(B * Tin).astype(jnp.int32)
    x = _sc_gather(tok_embds, ids)
    logits_t = _tc_matmul_t(x, pos_embds, lm_head_w.T, lm_head_b.reshape(1, V))
    return logits_t.T.reshape(B, Tin, V)


# speedup vs baseline: 2.2168x; 1.8452x over previous
import jax
import jax.numpy as jnp
from jax.experimental.layout import Format, Layout, with_layout_constraint
from jax.experimental import pallas as pl
from jax.experimental.pallas import tpu as pltpu
from jax.experimental.pallas import tpu_sc as plsc

T = 2048
C = 768
V = 50257
BV = 512
NBLK = (V + BV - 1) // BV
NCHUNK = 4
BPC = (NBLK + NCHUNK - 1) // NCHUNK


def _sc_gather(tok_embds, ids):
    mesh = plsc.VectorSubcoreMesh(core_axis_name="c", subcore_axis_name="s")
    nw = 32
    b_per_w = T // nw

    @pl.kernel(
        out_type=jax.ShapeDtypeStruct((T, C), tok_embds.dtype),
        mesh=mesh,
        scratch_types=[
            pltpu.VMEM((b_per_w,), jnp.int32),
            pltpu.VMEM((b_per_w, C), jnp.float32),
            pltpu.SemaphoreType.DMA,
        ],
    )
    def gather_kernel(table_hbm, idx_hbm, out_hbm, idx_v, rows_v, sem):
        wid = jax.lax.axis_index("s") * 2 + jax.lax.axis_index("c")
        base = wid * b_per_w
        pltpu.sync_copy(idx_hbm.at[pl.ds(base, b_per_w)], idx_v)
        pltpu.async_copy(table_hbm.at[idx_v], rows_v, sem).wait()
        pltpu.sync_copy(rows_v, out_hbm.at[pl.ds(base, b_per_w)])

    return gather_kernel(tok_embds, ids)


def _prep_body(x_ref, p_ref, o_ref):
    o_ref[...] = jnp.transpose((x_ref[...] + p_ref[...]).astype(jnp.bfloat16))


def _prep_xbt(x, pos):
    return pl.pallas_call(
        _prep_body,
        out_shape=jax.ShapeDtypeStruct((C, T), jnp.bfloat16),
    )(x, pos)


def _mm_body(xbt_ref, wt_ref, b_ref, o_ref):
    acc = jnp.dot(
        wt_ref[...].astype(jnp.bfloat16),
        xbt_ref[...],
        preferred_element_type=jnp.float32,
    )
    acc = acc + jnp.transpose(b_ref[...])
    o_ref[...] = acc.reshape(BV, 1, T)


def _mm_chunk(xbt, wt, b2, block_off, nblk, vc):
    return pl.pallas_call(
        _mm_body,
        grid=(nblk,),
        in_specs=[
            pl.BlockSpec((C, T), lambda j: (0, 0)),
            pl.BlockSpec((BV, C), lambda j: (block_off + j, 0)),
            pl.BlockSpec((1, BV), lambda j: (0, block_off + j)),
        ],
        out_specs=pl.BlockSpec((BV, 1, T), lambda j: (j, 0, 0)),
        out_shape=jax.ShapeDtypeStruct((vc, 1, T), jnp.float32),
        compiler_params=pltpu.CompilerParams(dimension_semantics=("arbitrary",)),
    )(xbt, wt, b2)


def kernel(inputs, tok_embds, pos_embds, lm_head_w, lm_head_b):
    B, Tin = inputs.shape
    ids = inputs.reshape(B * Tin).astype(jnp.int32)
    x = _sc_gather(tok_embds, ids)
    xbt = _prep_xbt(x, pos_embds)
    wt = lm_head_w.T
    b2 = lm_head_b.reshape(1, V)
    out3 = _mm_chunk(xbt, wt, b2, 0, NBLK, V)
    return jnp.transpose(out3, (1, 2, 0))


# speedup vs baseline: 2.2469x; 1.0136x over previous
import jax
import jax.numpy as jnp
from jax.experimental.layout import Format, Layout, with_layout_constraint
from jax.experimental import pallas as pl
from jax.experimental.pallas import tpu as pltpu
from jax.experimental.pallas import tpu_sc as plsc

T = 2048
C = 768
V = 50257
BV = 512
NBLK = (V + BV - 1) // BV
NCHUNK = 4
BPC = (NBLK + NCHUNK - 1) // NCHUNK


def _sc_gather(tok_embds, ids):
    mesh = plsc.VectorSubcoreMesh(core_axis_name="c", subcore_axis_name="s")
    nw = 32
    b_per_w = T // nw

    @pl.kernel(
        out_type=jax.ShapeDtypeStruct((T, C), tok_embds.dtype),
        mesh=mesh,
        scratch_types=[
            pltpu.VMEM((b_per_w,), jnp.int32),
            pltpu.VMEM((b_per_w, C), jnp.float32),
            pltpu.SemaphoreType.DMA,
        ],
    )
    def gather_kernel(table_hbm, idx_hbm, out_hbm, idx_v, rows_v, sem):
        wid = jax.lax.axis_index("s") * 2 + jax.lax.axis_index("c")
        base = wid * b_per_w
        pltpu.sync_copy(idx_hbm.at[pl.ds(base, b_per_w)], idx_v)
        pltpu.async_copy(table_hbm.at[idx_v], rows_v, sem).wait()
        pltpu.sync_copy(rows_v, out_hbm.at[pl.ds(base, b_per_w)])

    return gather_kernel(tok_embds, ids)


def _mm_body(x_ref, p_ref, wt_ref, b_ref, o_ref, xbt_ref):
    j = pl.program_id(0)

    @pl.when(j == 0)
    def _():
        xbt_ref[...] = jnp.transpose((x_ref[...] + p_ref[...]).astype(jnp.bfloat16))

    acc = jnp.dot(
        wt_ref[...].astype(jnp.bfloat16),
        xbt_ref[...],
        preferred_element_type=jnp.float32,
    )
    acc = acc + jnp.transpose(b_ref[...])
    o_ref[...] = acc.reshape(BV, 1, T)


def _mm_chunk(x, pos, wt, b2, block_off, nblk, vc):
    return pl.pallas_call(
        _mm_body,
        grid=(nblk,),
        in_specs=[
            pl.BlockSpec((T, C), lambda j: (0, 0)),
            pl.BlockSpec((T, C), lambda j: (0, 0)),
            pl.BlockSpec((BV, C), lambda j: (block_off + j, 0)),
            pl.BlockSpec((1, BV), lambda j: (0, block_off + j)),
        ],
        out_specs=pl.BlockSpec((BV, 1, T), lambda j: (j, 0, 0)),
        out_shape=jax.ShapeDtypeStruct((vc, 1, T), jnp.float32),
        scratch_shapes=[pltpu.VMEM((C, T), jnp.bfloat16)],
        compiler_params=pltpu.CompilerParams(dimension_semantics=("arbitrary",)),
    )(x, pos, wt, b2)


def kernel(inputs, tok_embds, pos_embds, lm_head_w, lm_head_b):
    B, Tin = inputs.shape
    ids = inputs.reshape(B * Tin).astype(jnp.int32)
    x = _sc_gather(tok_embds, ids)
    wt = lm_head_w.T
    b2 = lm_head_b.reshape(1, V)
    out3 = _mm_chunk(x, pos_embds, wt, b2, 0, NBLK, V)
    return jnp.transpose(out3, (1, 2, 0))


# speedup vs baseline: 2.4536x; 1.0920x over previous
import jax
import jax.numpy as jnp
from jax.experimental.layout import Format, Layout, with_layout_constraint
from jax.experimental import pallas as pl
from jax.experimental.pallas import tpu as pltpu
from jax.experimental.pallas import tpu_sc as plsc

T = 2048
C = 768
V = 50257
BV = 1024
NBLK = (V + BV - 1) // BV
NCHUNK = 4
BPC = (NBLK + NCHUNK - 1) // NCHUNK


def _sc_gather(tok_embds, ids):
    mesh = plsc.VectorSubcoreMesh(core_axis_name="c", subcore_axis_name="s")
    nw = 32
    b_per_w = T // nw

    @pl.kernel(
        out_type=jax.ShapeDtypeStruct((T, C), tok_embds.dtype),
        mesh=mesh,
        scratch_types=[
            pltpu.VMEM((b_per_w,), jnp.int32),
            pltpu.VMEM((b_per_w, C), jnp.float32),
            pltpu.SemaphoreType.DMA,
        ],
    )
    def gather_kernel(table_hbm, idx_hbm, out_hbm, idx_v, rows_v, sem):
        wid = jax.lax.axis_index("s") * 2 + jax.lax.axis_index("c")
        base = wid * b_per_w
        pltpu.sync_copy(idx_hbm.at[pl.ds(base, b_per_w)], idx_v)
        pltpu.async_copy(table_hbm.at[idx_v], rows_v, sem).wait()
        pltpu.sync_copy(rows_v, out_hbm.at[pl.ds(base, b_per_w)])

    return gather_kernel(tok_embds, ids)


def _mm_body(x_ref, p_ref, wt_ref, b_ref, o_ref, xbt_ref):
    j = pl.program_id(0)

    @pl.when(j == 0)
    def _():
        xbt_ref[...] = jnp.transpose((x_ref[...] + p_ref[...]).astype(jnp.bfloat16))

    acc = jnp.dot(
        wt_ref[...].astype(jnp.bfloat16),
        xbt_ref[...],
        preferred_element_type=jnp.float32,
    )
    acc = acc + jnp.transpose(b_ref[...])
    o_ref[...] = acc.reshape(BV, 1, T)


def _mm_chunk(x, pos, wt, b2, block_off, nblk, vc):
    return pl.pallas_call(
        _mm_body,
        grid=(nblk,),
        in_specs=[
            pl.BlockSpec((T, C), lambda j: (0, 0)),
            pl.BlockSpec((T, C), lambda j: (0, 0)),
            pl.BlockSpec((BV, C), lambda j: (block_off + j, 0)),
            pl.BlockSpec((1, BV), lambda j: (0, block_off + j)),
        ],
        out_specs=pl.BlockSpec((BV, 1, T), lambda j: (j, 0, 0)),
        out_shape=jax.ShapeDtypeStruct((vc, 1, T), jnp.float32),
        scratch_shapes=[pltpu.VMEM((C, T), jnp.bfloat16)],
        compiler_params=pltpu.CompilerParams(dimension_semantics=("arbitrary",)),
    )(x, pos, wt, b2)


def kernel(inputs, tok_embds, pos_embds, lm_head_w, lm_head_b):
    B, Tin = inputs.shape
    ids = inputs.reshape(B * Tin).astype(jnp.int32)
    x = _sc_gather(tok_embds, ids)
    wt = lm_head_w.T
    b2 = lm_head_b.reshape(1, V)
    out3 = _mm_chunk(x, pos_embds, wt, b2, 0, NBLK, V)
    return jnp.transpose(out3, (1, 2, 0))
